# baseline (speedup 1.0000x reference)
import functools

import jax
import jax.numpy as jnp
from jax import lax
from jax.experimental import pallas as pl
from jax.experimental.pallas import tpu as pltpu
from jax.experimental.pallas import tpu_sc as plsc

NC = 2
NS = 16
NW = NC * NS

R = 4096
K = 50
D = 128
RPW = R // NW
HC = 32
NCHUNK = K * (RPW // HC)
NBUF = 10
NGROUP = NCHUNK // NBUF


def _make_gather():
    mesh = plsc.VectorSubcoreMesh(core_axis_name="c", subcore_axis_name="s")

    @functools.partial(
        pl.kernel,
        mesh=mesh,
        out_type=jax.ShapeDtypeStruct((K, R, D), jnp.float32),
        scratch_types=[
            pltpu.VMEM((K, RPW), jnp.int32),
        ]
        + [pltpu.VMEM((HC, D), jnp.float32) for _ in range(NBUF)]
        + [pltpu.SemaphoreType.DMA for _ in range(2 * NBUF)],
    )
    def gather_kernel(idx_hbm, table_hbm, out_hbm, idx_v, *bufs_and_sems):
        bufs = bufs_and_sems[:NBUF]
        sem_g = bufs_and_sems[NBUF:2 * NBUF]
        sem_s = bufs_and_sems[2 * NBUF:]
        wid = lax.axis_index("s") * NC + lax.axis_index("c")
        r0 = wid * RPW
        pltpu.sync_copy(idx_hbm.at[:, pl.ds(r0, RPW)], idx_v)

        def gather(c, j):
            k = c % K
            h = c // K
            pltpu.async_copy(
                table_hbm.at[idx_v.at[k, pl.ds(h * HC, HC)]], bufs[j],
                sem_g[j])

        def store(c, j):
            k = c % K
            h = c // K
            pltpu.async_copy(
                bufs[j], out_hbm.at[k, pl.ds(r0 + h * HC, HC)], sem_s[j])

        for j in range(NBUF):
            gather(j, j)

        def body(g, carry):
            c0 = g * NBUF
            for j in range(NBUF):
                pltpu.make_async_copy(
                    table_hbm.at[idx_v.at[0, pl.ds(0, HC)]], bufs[j],
                    sem_g[j]).wait()
                store(c0 + j, j)
            @pl.when(g < NGROUP - 1)
            def _():
                for j in range(NBUF):
                    pltpu.make_async_copy(
                        bufs[j], out_hbm.at[0, pl.ds(r0, HC)], sem_s[j]).wait()
                    gather(c0 + NBUF + j, j)
            return carry

        lax.fori_loop(0, NGROUP, body, 0)
        for j in range(NBUF):
            pltpu.make_async_copy(
                bufs[j], out_hbm.at[0, pl.ds(r0, HC)], sem_s[j]).wait()

    return gather_kernel


_gather = _make_gather()


def kernel(gps_idx, table):
    idx_t = gps_idx.astype(jnp.int32).T
    out_t = _gather(idx_t, table)
    return jnp.transpose(out_t, (1, 0, 2))


# speedup vs baseline: 1.0633x; 1.0633x over previous
import functools

import jax
import jax.numpy as jnp
from jax import lax
from jax.experimental import pallas as pl
from jax.experimental.pallas import tpu as pltpu
from jax.experimental.pallas import tpu_sc as plsc

NC = 2
NS = 16
NW = NC * NS

R = 4096
K = 50
D = 128
RPW = R // NW
HC = 64
NCHUNK = K * (RPW // HC)
NBUF = 10
NGROUP = NCHUNK // NBUF


def _make_gather():
    mesh = plsc.VectorSubcoreMesh(core_axis_name="c", subcore_axis_name="s")

    @functools.partial(
        pl.kernel,
        mesh=mesh,
        out_type=jax.ShapeDtypeStruct((K, R, D), jnp.float32),
        scratch_types=[
            pltpu.VMEM((K, RPW), jnp.int32),
        ]
        + [pltpu.VMEM((HC, D), jnp.float32) for _ in range(NBUF)]
        + [pltpu.SemaphoreType.DMA for _ in range(2 * NBUF)],
    )
    def gather_kernel(idx_hbm, table_hbm, out_hbm, idx_v, *bufs_and_sems):
        bufs = bufs_and_sems[:NBUF]
        sem_g = bufs_and_sems[NBUF:2 * NBUF]
        sem_s = bufs_and_sems[2 * NBUF:]
        wid = lax.axis_index("s") * NC + lax.axis_index("c")
        r0 = wid * RPW
        pltpu.sync_copy(idx_hbm.at[:, pl.ds(r0, RPW)], idx_v)

        def gather(c, j):
            k = c % K
            h = c // K
            pltpu.async_copy(
                table_hbm.at[idx_v.at[k, pl.ds(h * HC, HC)]], bufs[j],
                sem_g[j])

        def store(c, j):
            k = c % K
            h = c // K
            pltpu.async_copy(
                bufs[j], out_hbm.at[k, pl.ds(r0 + h * HC, HC)], sem_s[j])

        for j in range(NBUF):
            gather(j, j)

        def body(g, carry):
            c0 = g * NBUF
            for j in range(NBUF):
                pltpu.make_async_copy(
                    table_hbm.at[idx_v.at[0, pl.ds(0, HC)]], bufs[j],
                    sem_g[j]).wait()
                store(c0 + j, j)
            @pl.when(g < NGROUP - 1)
            def _():
                for j in range(NBUF):
                    pltpu.make_async_copy(
                        bufs[j], out_hbm.at[0, pl.ds(r0, HC)], sem_s[j]).wait()
                    gather(c0 + NBUF + j, j)
            return carry

        lax.fori_loop(0, NGROUP, body, 0)
        for j in range(NBUF):
            pltpu.make_async_copy(
                bufs[j], out_hbm.at[0, pl.ds(r0, HC)], sem_s[j]).wait()

    return gather_kernel


_gather = _make_gather()


def kernel(gps_idx, table):
    idx_t = gps_idx.astype(jnp.int32).T
    out_t = _gather(idx_t, table)
    return jnp.transpose(out_t, (1, 0, 2))


# speedup vs baseline: 1.0670x; 1.0034x over previous
import functools

import jax
import jax.numpy as jnp
from jax import lax
from jax.experimental import pallas as pl
from jax.experimental.pallas import tpu as pltpu
from jax.experimental.pallas import tpu_sc as plsc

NC = 2
NS = 16
NW = NC * NS

R = 4096
K = 50
D = 128
RPW = R // NW
HC = 64
NCHUNK = K * (RPW // HC)
NBUF = 10
NGROUP = NCHUNK // NBUF


def _make_gather():
    mesh = plsc.VectorSubcoreMesh(core_axis_name="c", subcore_axis_name="s")

    @functools.partial(
        pl.kernel,
        mesh=mesh,
        out_type=jax.ShapeDtypeStruct((K, R, D), jnp.float32),
        scratch_types=[
            pltpu.VMEM((K, RPW), jnp.int32),
        ]
        + [pltpu.VMEM((HC, D), jnp.float32) for _ in range(NBUF)]
        + [pltpu.SemaphoreType.DMA for _ in range(2 * NBUF)],
    )
    def gather_kernel(idx_hbm, table_hbm, out_hbm, idx_v, *bufs_and_sems):
        bufs = bufs_and_sems[:NBUF]
        sem_g = bufs_and_sems[NBUF:2 * NBUF]
        sem_s = bufs_and_sems[2 * NBUF:]
        wid = lax.axis_index("s") * NC + lax.axis_index("c")
        r0 = wid * RPW
        pltpu.sync_copy(idx_hbm.at[:, pl.ds(r0, RPW)], idx_v)

        def gather_desc(c, j):
            k = c % K
            h = c // K
            return pltpu.make_async_copy(
                table_hbm.at[idx_v.at[k, pl.ds(h * HC, HC)]], bufs[j],
                sem_g[j])

        def store_desc(c, j):
            k = c % K
            h = c // K
            return pltpu.make_async_copy(
                bufs[j], out_hbm.at[k, pl.ds(r0 + h * HC, HC)], sem_s[j])

        for j in range(NBUF):
            gather_desc(j, j).start()

        def body(g, carry):
            c0 = g * NBUF
            for j in range(NBUF):
                gather_desc(c0 + j, j).wait()
                store_desc(c0 + j, j).start()
            @pl.when(g < NGROUP - 1)
            def _():
                for j in range(NBUF):
                    store_desc(c0 + j, j).wait()
                    gather_desc(c0 + NBUF + j, j).start()
            return carry

        lax.fori_loop(0, NGROUP, body, 0)
        for j in range(NBUF):
            store_desc((NGROUP - 1) * NBUF + j, j).wait()

    return gather_kernel


_gather = _make_gather()


def kernel(gps_idx, table):
    idx_t = gps_idx.astype(jnp.int32).T
    out_t = _gather(idx_t, table)
    return jnp.transpose(out_t, (1, 0, 2))
